# baseline (speedup 1.0000x reference)
import functools

import jax
import jax.numpy as jnp
from jax import lax
from jax.experimental import pallas as pl
from jax.experimental.pallas import tpu as pltpu
from jax.experimental.pallas import tpu_sc as plsc

HID = 64
NC, NS, L = 2, 16, 16
NW = NC * NS

B, F, W, PL = 1024, 26, 4, 10
WALK = W * PL
M = B * F
ROWS_PER_W = M // NW

NB1 = 8
ROWS_B = F * WALK
K1_CHUNKS = B // NW // NB1
NGRP1 = NB1 * ROWS_B // L
GRP_B = ROWS_B // L
OUT_COLS = 128
CHUNK_ROWS = NB1 * ROWS_B // OUT_COLS
PAD_ROWS = 72
R_TOT = NW * K1_CHUNKS * PAD_ROWS

C = 13
IDX_C = C * WALK
SUB_CHUNKS = NB1 * ROWS_B // IDX_C
CHUNKS = K1_CHUNKS * SUB_CHUNKS
NBUF = 2

_mesh = plsc.VectorSubcoreMesh(core_axis_name="c", subcore_axis_name="s")


def _make_flatten():
  @functools.partial(
      pl.kernel,
      mesh=_mesh,
      compiler_params=pltpu.CompilerParams(
          use_tc_tiling_on_sc=True, needs_layout_passes=False),
      out_type=jax.ShapeDtypeStruct((R_TOT, OUT_COLS), jnp.int32),
      scratch_types=[
          pltpu.VMEM((NB1, F, W, PL), jnp.int32),
          pltpu.VMEM((PAD_ROWS, OUT_COLS), jnp.int32),
          [pltpu.VMEM((GRP_B * L,), jnp.int32) for _ in range(3)],
          pltpu.VMEM((NB1 * L,), jnp.int32),
      ],
  )
  def body(idx_hbm, out_hbm, slab, flat, coord_tabs, btab):
    wid = lax.axis_index("s") * NC + lax.axis_index("c")
    b_base = wid * (B // NW)
    ftab, wtab, ptab = coord_tabs

    iota = lax.iota(jnp.int32, L)
    for g in range(GRP_B):
      kk = iota + g * L
      f = (kk.astype(jnp.float32) * (1.0 / WALK)).astype(jnp.int32)
      t = kk - f * WALK
      w = (t.astype(jnp.float32) * (1.0 / PL)).astype(jnp.int32)
      p = t - w * PL
      ftab[pl.ds(g * L, L)] = f
      wtab[pl.ds(g * L, L)] = w
      ptab[pl.ds(g * L, L)] = p
    for bloc in range(NB1):
      btab[pl.ds(bloc * L, L)] = iota * 0 + bloc

    def chunk(ci, _):
      b0 = b_base + ci * NB1
      pltpu.sync_copy(idx_hbm.at[pl.ds(b0, NB1)], slab)
      for g in range(NGRP1):
        bloc, gb = g // GRP_B, g % GRP_B
        bv = btab[pl.ds(bloc * L, L)]
        f = ftab[pl.ds(gb * L, L)]
        w = wtab[pl.ds(gb * L, L)]
        p = ptab[pl.ds(gb * L, L)]
        flat[(g * L) // OUT_COLS,
             pl.ds((g * L) % OUT_COLS, L)] = plsc.load_gather(
                 slab, [bv, f, w, p])
      out_r0 = (wid * K1_CHUNKS + ci) * PAD_ROWS
      pltpu.sync_copy(flat, out_hbm.at[pl.ds(out_r0, PAD_ROWS), :])
      return 0

    lax.fori_loop(0, K1_CHUNKS, chunk, 0)

  return body


def _make_gather_reduce():
  @functools.partial(
      pl.kernel,
      mesh=_mesh,
      compiler_params=pltpu.CompilerParams(
          use_tc_tiling_on_sc=False, needs_layout_passes=False),
      out_type=jax.ShapeDtypeStruct((M, HID), jnp.float32),
      scratch_types=[
          [pltpu.VMEM((IDX_C,), jnp.int32) for _ in range(NBUF)],
          [pltpu.VMEM((IDX_C, HID), jnp.float32) for _ in range(NBUF)],
          [pltpu.VMEM((C, HID), jnp.float32) for _ in range(NBUF)],
          [pltpu.SemaphoreType.DMA for _ in range(NBUF)],
          [pltpu.SemaphoreType.DMA for _ in range(NBUF)],
      ],
  )
  def body(idx_hbm, table_hbm, out_hbm, idx_bufs, rows_bufs, acc_bufs,
           gsems, osems):
    wid = lax.axis_index("s") * NC + lax.axis_index("c")
    row_base = wid * ROWS_PER_W
    flat_base = wid * (K1_CHUNKS * PAD_ROWS * OUT_COLS)

    def split(ci):
      return ci >> 4, ci & (SUB_CHUNKS - 1)

    def start_gather(ci, bf):
      j, r = split(ci)
      pltpu.sync_copy(
          idx_hbm.at[pl.ds(
              flat_base + j * (PAD_ROWS * OUT_COLS) + r * IDX_C, IDX_C)],
          idx_bufs[bf])
      pltpu.async_copy(table_hbm.at[idx_bufs[bf]], rows_bufs[bf], gsems[bf])

    def wait_gather(bf):
      pltpu.make_async_copy(
          table_hbm.at[idx_bufs[bf]], rows_bufs[bf], gsems[bf]).wait()

    def out_slice(ci):
      j, r = split(ci)
      return out_hbm.at[pl.ds(row_base + (j * SUB_CHUNKS + r) * C, C), :]

    start_gather(0, 0)

    def outer(ci2, _):
      base_ci = ci2 * NBUF
      for bf in range(NBUF):
        ci = base_ci + bf
        nbf = (bf + 1) % NBUF

        @pl.when(ci + 1 < CHUNKS)
        def _():
          start_gather(ci + 1, nbf)

        wait_gather(bf)
        rows_v = rows_bufs[bf]
        acc_v = acc_bufs[bf]

        @pl.when(ci2 > 0)
        def _():
          pltpu.make_async_copy(acc_v, out_slice(ci), osems[bf]).wait()

        for r in range(C):
          def red_body(jo, carry):
            a0, a1, a2, a3 = carry
            for ji in range(4):
              rr = r * WALK + jo * 4 + ji
              a0 = a0 + rows_v[rr, pl.ds(0, L)]
              a1 = a1 + rows_v[rr, pl.ds(L, L)]
              a2 = a2 + rows_v[rr, pl.ds(2 * L, L)]
              a3 = a3 + rows_v[rr, pl.ds(3 * L, L)]
            return (a0, a1, a2, a3)

          z = jnp.zeros((L,), jnp.float32)
          a0, a1, a2, a3 = lax.fori_loop(0, WALK // 4, red_body,
                                         (z, z, z, z))
          acc_v[r, pl.ds(0, L)] = a0
          acc_v[r, pl.ds(L, L)] = a1
          acc_v[r, pl.ds(2 * L, L)] = a2
          acc_v[r, pl.ds(3 * L, L)] = a3
        pltpu.async_copy(acc_v, out_slice(ci), osems[bf])
      return 0

    lax.fori_loop(0, CHUNKS // NBUF, outer, 0)
    for bf in range(NBUF):
      pltpu.make_async_copy(
          acc_bufs[bf], out_slice(CHUNKS - NBUF + bf), osems[bf]).wait()

  return body


_flatten = _make_flatten()
_gather_reduce = _make_gather_reduce()


def kernel(walk_paths, node_embeddings, linear_w):
  del linear_w
  flat_idx = _flatten(walk_paths).reshape(R_TOT * OUT_COLS)
  out = _gather_reduce(flat_idx, node_embeddings)
  return out.reshape(B, F, HID)
